# baseline (speedup 1.0000x reference)
import functools

import jax
import jax.numpy as jnp
from jax import lax
from jax.experimental import pallas as pl
from jax.experimental.pallas import tpu as pltpu
from jax.experimental.pallas import tpu_sc as plsc


@functools.lru_cache(maxsize=None)
def _make_gather(num_rows, dim, n_r, n_c):
    info = plsc.get_sparse_core_info()
    nc, ns, nl = info.num_cores, info.num_subcores, info.num_lanes
    nw = nc * ns
    assert n_r % (nw * nl) == 0
    r_per_w = n_r // nw
    b_per_w = r_per_w * n_c
    batch = n_r * n_c
    chunk = 1024
    while b_per_w % chunk:
        chunk //= 2
    n_chunks = b_per_w // chunk

    mesh = plsc.VectorSubcoreMesh(core_axis_name="c", subcore_axis_name="s")

    @functools.partial(
        pl.kernel,
        mesh=mesh,
        out_type=jax.ShapeDtypeStruct((batch, dim), jnp.float32),
        scratch_types=[
            pltpu.VMEM((n_c, r_per_w), jnp.int32),
            pltpu.VMEM((b_per_w,), jnp.int32),
            pltpu.VMEM((2, chunk, dim), jnp.float32),
            pltpu.SemaphoreType.DMA,
            pltpu.SemaphoreType.DMA,
        ],
        compiler_params=pltpu.CompilerParams(
            use_tc_tiling_on_sc=False, needs_layout_passes=False),
    )
    def gather(idxt_hbm, table_hbm, out_hbm, idx_blk, idx_v, rows_v, gsem, ssem):
        wid = lax.axis_index("s") * nc + lax.axis_index("c")
        r0 = wid * r_per_w
        base = wid * b_per_w
        pltpu.sync_copy(idxt_hbm.at[:, pl.ds(r0, r_per_w)], idx_blk)
        step = lax.iota(jnp.int32, nl) * n_c
        for c in range(n_c):
            row = idx_blk.at[c]
            for i in range(r_per_w // nl):
                vals = row[pl.ds(i * nl, nl)]
                plsc.store_scatter(idx_v, [step + (i * nl * n_c + c)], vals)

        def gather_start(cnk):
            return pltpu.async_copy(
                table_hbm.at[idx_v.at[pl.ds(cnk * chunk, chunk)]],
                rows_v.at[cnk % 2], gsem)

        def store_start(cnk):
            return pltpu.async_copy(
                rows_v.at[cnk % 2],
                out_hbm.at[pl.ds(base + cnk * chunk, chunk)], ssem)

        g = gather_start(0)
        s_prev = None
        for cnk in range(n_chunks):
            g.wait()
            s = store_start(cnk)
            if cnk + 1 < n_chunks:
                if s_prev is not None:
                    s_prev.wait()
                g = gather_start(cnk + 1)
            s_prev_old, s_prev = s_prev, s
        s_prev.wait()
        if n_chunks > 1:
            s_prev_old.wait()

    return gather


def kernel(indices, weight):
    n_r, n_c = indices.shape
    dim = weight.shape[1]
    gather = _make_gather(weight.shape[0], dim, n_r, n_c)
    out = gather(indices.T.astype(jnp.int32), weight)
    return out.reshape(n_r, n_c, dim)
